# Initial kernel scaffold; baseline (speedup 1.0000x reference)
import jax
import jax.numpy as jnp
from jax.experimental import pallas as pl


def kernel(Corpus_, batch_inputs, entity_embeddings, edge_list, W, a, W_out, a_out):
    raise NotImplementedError("write your pallas kernel here")


# speedup vs baseline: 16.5731x; 16.5731x over previous
import functools

import jax
import jax.numpy as jnp
from jax import lax
from jax.experimental import pallas as pl
from jax.experimental.pallas import tpu as pltpu
from jax.experimental.pallas import tpu_sc as plsc

N = 10000
E = 320000
NF = 128
NHID = 16
NHEADS = 8
ALPHA = 0.2
D = NHEADS * NHID

NC = 2
NS = 16
DH = D // NC
G = DH // 16
TW = 16 + DH
KB = 80
NB = 256
EPT = NB * KB
EPAD = NS * EPT
NPAD = 10112
RPT = NPAD // NS


def _elu(x):
    return jnp.where(x > 0.0, x, jnp.exp(jnp.minimum(x, 0.0)) - 1.0)


def _mm_a_body(x_ref, wc_ref, p1_ref, p2_ref, th_ref, s1_ref):
    x = x_ref[...]
    h = jnp.dot(x, wc_ref[...], preferred_element_type=jnp.float32)
    s2 = jnp.dot(x, p2_ref[...], preferred_element_type=jnp.float32)
    th_ref[0] = jnp.concatenate([s2[:, :16], h[:, :DH]], axis=1)
    th_ref[1] = jnp.concatenate([s2[:, 16:], h[:, DH:]], axis=1)
    s1 = jnp.dot(x, p1_ref[...], preferred_element_type=jnp.float32)
    s1_ref[0] = s1[:, :16]
    s1_ref[1] = s1[:, 16:]


def _mm_b_body(acc_ref, r_ref, wo_ref, q1_ref, q2_ref, th_ref, s1_ref):
    aw = acc_ref[0][:, :16]
    ah = jnp.concatenate([acc_ref[0][:, 16:], acc_ref[1][:, 16:]], axis=1)
    den = jnp.dot(aw, r_ref[...], preferred_element_type=jnp.float32) + 1e-16
    x1 = _elu(ah / den)
    h2 = jnp.dot(x1, wo_ref[...], preferred_element_type=jnp.float32)
    s2 = jnp.dot(x1, q2_ref[...], preferred_element_type=jnp.float32)
    th_ref[0] = jnp.concatenate([s2[:, :16], h2[:, :DH]], axis=1)
    th_ref[1] = jnp.concatenate([s2[:, 16:], h2[:, DH:]], axis=1)
    s1 = jnp.dot(x1, q1_ref[...], preferred_element_type=jnp.float32)
    s1_ref[0] = s1[:, :16]
    s1_ref[1] = s1[:, 16:]


def _mm_c_body(acc_ref, r_ref, out_ref):
    aw = acc_ref[0][:, :16]
    ah = jnp.concatenate([acc_ref[0][:, 16:], acc_ref[1][:, 16:]], axis=1)
    den = jnp.dot(aw, r_ref[...], preferred_element_type=jnp.float32) + 1e-16
    out_ref[...] = _elu(ah / den)


_RB = 632


def _mm_a(xp, wc, p1, p2):
    g = NPAD // _RB
    return pl.pallas_call(
        _mm_a_body,
        grid=(g,),
        in_specs=[
            pl.BlockSpec((_RB, NF), lambda i: (i, 0)),
            pl.BlockSpec((NF, D), lambda i: (0, 0)),
            pl.BlockSpec((NF, 32), lambda i: (0, 0)),
            pl.BlockSpec((NF, 32), lambda i: (0, 0)),
        ],
        out_specs=[
            pl.BlockSpec((NC, _RB, TW), lambda i: (0, i, 0)),
            pl.BlockSpec((NC, _RB, 16), lambda i: (0, i, 0)),
        ],
        out_shape=[
            jax.ShapeDtypeStruct((NC, NPAD, TW), jnp.float32),
            jax.ShapeDtypeStruct((NC, NPAD, 16), jnp.float32),
        ],
    )(xp, wc, p1, p2)


def _mm_b(acc, r, wo, q1, q2):
    g = NPAD // _RB
    return pl.pallas_call(
        _mm_b_body,
        grid=(g,),
        in_specs=[
            pl.BlockSpec((NC, _RB, TW), lambda i: (0, i, 0)),
            pl.BlockSpec((16, D), lambda i: (0, 0)),
            pl.BlockSpec((D, D), lambda i: (0, 0)),
            pl.BlockSpec((D, 32), lambda i: (0, 0)),
            pl.BlockSpec((D, 32), lambda i: (0, 0)),
        ],
        out_specs=[
            pl.BlockSpec((NC, _RB, TW), lambda i: (0, i, 0)),
            pl.BlockSpec((NC, _RB, 16), lambda i: (0, i, 0)),
        ],
        out_shape=[
            jax.ShapeDtypeStruct((NC, NPAD, TW), jnp.float32),
            jax.ShapeDtypeStruct((NC, NPAD, 16), jnp.float32),
        ],
    )(acc, r, wo, q1, q2)


def _mm_c(acc, r):
    g = NPAD // _RB
    return pl.pallas_call(
        _mm_c_body,
        grid=(g,),
        in_specs=[
            pl.BlockSpec((NC, _RB, TW), lambda i: (0, i, 0)),
            pl.BlockSpec((16, D), lambda i: (0, 0)),
        ],
        out_specs=pl.BlockSpec((_RB, D), lambda i: (i, 0)),
        out_shape=jax.ShapeDtypeStruct((NPAD, D), jnp.float32),
    )(acc, r)


@functools.cache
def _make_edge_kernel():
    mesh = plsc.VectorSubcoreMesh(
        core_axis_name="c", subcore_axis_name="s",
        num_cores=NC, num_subcores=NS)

    out_type = pltpu.HBM((NC, NPAD, TW), jnp.float32)
    scratch_types = [
        pltpu.VMEM((NB, KB), jnp.int32),
        pltpu.VMEM((NB, KB), jnp.int32),
        pltpu.VMEM((2, KB), jnp.int32),
        pltpu.VMEM((2, KB), jnp.int32),
        pltpu.VMEM((2 * KB, 16), jnp.float32),
        pltpu.VMEM((2 * KB, TW), jnp.float32),
        pltpu.VMEM((2 * KB, TW), jnp.float32),
        pltpu.VMEM_SHARED((NPAD, TW), jnp.float32),
        pltpu.SemaphoreType.DMA,
        pltpu.SemaphoreType.DMA,
        pltpu.SemaphoreType.DMA,
        pltpu.SemaphoreType.DMA,
    ]

    @functools.partial(pl.kernel, out_type=out_type, mesh=mesh,
                       scratch_types=scratch_types,
                       compiler_params=pltpu.CompilerParams(
                           use_tc_tiling_on_sc=False))
    def edge_kernel(s1t, tht, srcg, dstg, outa,
                    src_v, dst_v, d2b, s2i, s1b, thb, ob,
                    acc, g0, g1, t0, t1):
        c = lax.axis_index("c")
        s = lax.axis_index("s")
        gsem = (g0, g1)
        tsem = (t0, t1)

        def zb(i, _):
            for kk in range(TW // 16):
                ob[i, pl.ds(16 * kk, 16)] = jnp.zeros((16,), jnp.float32)
            return 0
        lax.fori_loop(0, KB, zb, 0)
        base = s * RPT
        nfull = RPT // KB
        rem = RPT - nfull * KB
        for i in range(nfull):
            pltpu.sync_copy(ob.at[pl.ds(0, KB)],
                            acc.at[pl.ds(base + KB * i, KB)])
        if rem:
            pltpu.sync_copy(ob.at[pl.ds(0, rem)],
                            acc.at[pl.ds(base + KB * nfull, rem)])
        plsc.subcore_barrier()

        pltpu.sync_copy(srcg.at[s], src_v)
        pltpu.sync_copy(dstg.at[s], dst_v)
        off = c * NPAD

        def issue_g(b, slot):
            sl = pl.ds(slot * KB, KB)
            for kk in range(KB // 16):
                ds16 = pl.ds(16 * kk, 16)
                d2b[slot, ds16] = dst_v[b, ds16] + off
                s2i[slot, ds16] = src_v[b, ds16] + off
            pltpu.async_copy(s1t.at[s2i.at[slot]], s1b.at[sl], gsem[slot])
            pltpu.async_copy(tht.at[d2b.at[slot]], thb.at[sl], gsem[slot])

        def wait_g(slot):
            sl = pl.ds(slot * KB, KB)
            pltpu.make_async_copy(s1t.at[s2i.at[0]], s1b.at[sl],
                                  gsem[slot]).wait()
            pltpu.make_async_copy(tht.at[d2b.at[0]], thb.at[sl],
                                  gsem[slot]).wait()

        def issue_s(b, slot):
            sl = pl.ds(slot * KB, KB)
            pltpu.async_copy(ob.at[sl], acc.at[src_v.at[b]], tsem[slot],
                             add=True)

        def wait_s(slot):
            sl = pl.ds(slot * KB, KB)
            pltpu.make_async_copy(ob.at[sl], acc.at[src_v.at[0]],
                                  tsem[slot]).wait()

        def compute(slot):
            @plsc.parallel_loop(0, KB, unroll=8)
            def _(j):
                row = slot * KB + j
                sc_ = s1b[row, :] + thb[row, pl.ds(0, 16)]
                w = jnp.exp(-jnp.where(sc_ >= 0.0, sc_, ALPHA * sc_))
                ob[row, pl.ds(0, 16)] = w
                for kk in range(G):
                    sl16 = pl.ds(16 + 16 * kk, 16)
                    ob[row, sl16] = w[kk] * thb[row, sl16]

        issue_g(0, 0)

        def obody(i, _):
            b0 = 2 * i
            b1 = 2 * i + 1
            issue_g(b1, 1)
            wait_g(0)

            @pl.when(i > 0)
            def _():
                wait_s(0)
            compute(0)
            issue_s(b0, 0)

            @pl.when(i < NB // 2 - 1)
            def _():
                issue_g(b0 + 2, 0)
            wait_g(1)

            @pl.when(i > 0)
            def _():
                wait_s(1)
            compute(1)
            issue_s(b1, 1)
            return 0

        lax.fori_loop(0, NB // 2, obody, 0)
        wait_s(0)
        wait_s(1)
        plsc.subcore_barrier()

        pltpu.sync_copy(acc.at[pl.ds(base, RPT)],
                        outa.at[c, pl.ds(base, RPT)])

    return edge_kernel


def kernel(Corpus_, batch_inputs, entity_embeddings, edge_list, W, a,
           W_out, a_out):
    f32 = jnp.float32
    wc = jnp.transpose(W, (1, 0, 2)).reshape(NF, D)
    p1 = jnp.einsum("hfj,hj->fh", W, a[:, 0, :NHID])
    p2 = jnp.einsum("hfj,hj->fh", W, a[:, 0, NHID:])
    z8 = jnp.zeros((NF, 8), f32)
    z12 = jnp.zeros((NF, 12), f32)
    p1 = jnp.concatenate([p1, z8, p1[:, 4:8], z12], axis=1)
    p2 = jnp.concatenate([p2, z8, p2[:, 4:8], z12], axis=1)
    q1 = W_out @ a_out[0, :D]
    q2 = W_out @ a_out[0, D:]
    zq = jnp.zeros((D, 8), f32)
    zq12 = jnp.zeros((D, 12), f32)
    q1t = jnp.tile(q1[:, None], (1, 8))
    q2t = jnp.tile(q2[:, None], (1, 8))
    q1 = jnp.concatenate([q1t, zq, q1t[:, :4], zq12], axis=1)
    q2 = jnp.concatenate([q2t, zq, q2t[:, :4], zq12], axis=1)
    r = (jnp.arange(D)[None, :] // NHID ==
         jnp.arange(16)[:, None]).astype(f32)

    xp = jnp.pad(entity_embeddings, ((0, NPAD - N), (0, 0)))
    src = edge_list[0].astype(jnp.int32)
    dst = edge_list[1].astype(jnp.int32)
    padv = jnp.full((EPAD - E,), N, jnp.int32)
    srcg = jnp.concatenate([src, padv]).reshape(NS, NB, KB)
    dstg = jnp.concatenate([dst, padv]).reshape(NS, NB, KB)

    ek = _make_edge_kernel()
    th1, s1 = _mm_a(xp, wc, p1, p2)
    acc1 = ek(s1.reshape(NC * NPAD, 16), th1.reshape(NC * NPAD, TW),
              srcg, dstg)
    th2, s1b = _mm_b(acc1, r, W_out, q1, q2)
    acc2 = ek(s1b.reshape(NC * NPAD, 16), th2.reshape(NC * NPAD, TW),
              srcg, dstg)
    out = _mm_c(acc2, r)
    return out[:N]


# speedup vs baseline: 17.8276x; 1.0757x over previous
import functools

import jax
import jax.numpy as jnp
from jax import lax
from jax.experimental import pallas as pl
from jax.experimental.pallas import tpu as pltpu
from jax.experimental.pallas import tpu_sc as plsc

N = 10000
E = 320000
NF = 128
NHID = 16
NHEADS = 8
ALPHA = 0.2
D = NHEADS * NHID

NC = 2
NS = 16
DH = D // NC
G = DH // 16
KB = 64
NB = 320
EPT = NB * KB
EPAD = NS * EPT
NPAD = 10112
RPT = NPAD // NS


def _elu(x):
    return jnp.where(x > 0.0, x, jnp.exp(jnp.minimum(x, 0.0)) - 1.0)


def _mm_a_body(x_ref, wc_ref, p1_ref, p2_ref, h_ref, s1_ref, s2_ref):
    x = x_ref[...]
    h = jnp.dot(x, wc_ref[...], preferred_element_type=jnp.float32)
    h_ref[0] = h[:, :DH]
    h_ref[1] = h[:, DH:]
    s1 = jnp.dot(x, p1_ref[...], preferred_element_type=jnp.float32)
    s1_ref[0] = s1[:, :16]
    s1_ref[1] = s1[:, 16:]
    s2 = jnp.dot(x, p2_ref[...], preferred_element_type=jnp.float32)
    s2_ref[0] = s2[:, :16]
    s2_ref[1] = s2[:, 16:]


def _mm_b_body(aw_ref, ah_ref, r_ref, wo_ref, q1_ref, q2_ref,
               h2_ref, s1_ref, s2_ref):
    aw = aw_ref[0]
    ah = jnp.concatenate([ah_ref[0], ah_ref[1]], axis=1)
    den = jnp.dot(aw, r_ref[...], preferred_element_type=jnp.float32) + 1e-16
    x1 = _elu(ah / den)
    h2 = jnp.dot(x1, wo_ref[...], preferred_element_type=jnp.float32)
    h2_ref[0] = h2[:, :DH]
    h2_ref[1] = h2[:, DH:]
    s1 = jnp.dot(x1, q1_ref[...], preferred_element_type=jnp.float32)
    s1_ref[0] = s1[:, :16]
    s1_ref[1] = s1[:, 16:]
    s2 = jnp.dot(x1, q2_ref[...], preferred_element_type=jnp.float32)
    s2_ref[0] = s2[:, :16]
    s2_ref[1] = s2[:, 16:]


def _mm_c_body(aw_ref, ah_ref, r_ref, out_ref):
    aw = aw_ref[0]
    ah = jnp.concatenate([ah_ref[0], ah_ref[1]], axis=1)
    den = jnp.dot(aw, r_ref[...], preferred_element_type=jnp.float32) + 1e-16
    out_ref[...] = _elu(ah / den)


_RB = 632


def _mm_a(xp, wc, p1, p2):
    g = NPAD // _RB
    return pl.pallas_call(
        _mm_a_body,
        grid=(g,),
        in_specs=[
            pl.BlockSpec((_RB, NF), lambda i: (i, 0)),
            pl.BlockSpec((NF, D), lambda i: (0, 0)),
            pl.BlockSpec((NF, 32), lambda i: (0, 0)),
            pl.BlockSpec((NF, 32), lambda i: (0, 0)),
        ],
        out_specs=[
            pl.BlockSpec((NC, _RB, DH), lambda i: (0, i, 0)),
            pl.BlockSpec((NC, _RB, 16), lambda i: (0, i, 0)),
            pl.BlockSpec((NC, _RB, 16), lambda i: (0, i, 0)),
        ],
        out_shape=[
            jax.ShapeDtypeStruct((NC, NPAD, DH), jnp.float32),
            jax.ShapeDtypeStruct((NC, NPAD, 16), jnp.float32),
            jax.ShapeDtypeStruct((NC, NPAD, 16), jnp.float32),
        ],
    )(xp, wc, p1, p2)


def _mm_b(aw, ah, r, wo, q1, q2):
    g = NPAD // _RB
    return pl.pallas_call(
        _mm_b_body,
        grid=(g,),
        in_specs=[
            pl.BlockSpec((NC, _RB, 16), lambda i: (0, i, 0)),
            pl.BlockSpec((NC, _RB, DH), lambda i: (0, i, 0)),
            pl.BlockSpec((16, D), lambda i: (0, 0)),
            pl.BlockSpec((D, D), lambda i: (0, 0)),
            pl.BlockSpec((D, 32), lambda i: (0, 0)),
            pl.BlockSpec((D, 32), lambda i: (0, 0)),
        ],
        out_specs=[
            pl.BlockSpec((NC, _RB, DH), lambda i: (0, i, 0)),
            pl.BlockSpec((NC, _RB, 16), lambda i: (0, i, 0)),
            pl.BlockSpec((NC, _RB, 16), lambda i: (0, i, 0)),
        ],
        out_shape=[
            jax.ShapeDtypeStruct((NC, NPAD, DH), jnp.float32),
            jax.ShapeDtypeStruct((NC, NPAD, 16), jnp.float32),
            jax.ShapeDtypeStruct((NC, NPAD, 16), jnp.float32),
        ],
    )(aw, ah, r, wo, q1, q2)


def _mm_c(aw, ah, r):
    g = NPAD // _RB
    return pl.pallas_call(
        _mm_c_body,
        grid=(g,),
        in_specs=[
            pl.BlockSpec((NC, _RB, 16), lambda i: (0, i, 0)),
            pl.BlockSpec((NC, _RB, DH), lambda i: (0, i, 0)),
            pl.BlockSpec((16, D), lambda i: (0, 0)),
        ],
        out_specs=pl.BlockSpec((_RB, D), lambda i: (i, 0)),
        out_shape=jax.ShapeDtypeStruct((NPAD, D), jnp.float32),
    )(aw, ah, r)


@functools.cache
def _make_edge_kernel():
    mesh = plsc.VectorSubcoreMesh(
        core_axis_name="c", subcore_axis_name="s",
        num_cores=NC, num_subcores=NS)

    out_type = [
        pltpu.HBM((NC, NPAD, 16), jnp.float32),
        pltpu.HBM((NC, NPAD, DH), jnp.float32),
    ]
    scratch_types = [
        pltpu.VMEM((NB, KB), jnp.int32),
        pltpu.VMEM((NB, KB), jnp.int32),
        pltpu.VMEM((2, KB), jnp.int32),
        pltpu.VMEM((2, KB), jnp.int32),
        pltpu.VMEM((2 * KB, 16), jnp.float32),
        pltpu.VMEM((2 * KB, 16), jnp.float32),
        pltpu.VMEM((2 * KB, DH), jnp.float32),
        pltpu.VMEM((2 * KB, 16), jnp.float32),
        pltpu.VMEM((2 * KB, DH), jnp.float32),
        pltpu.VMEM_SHARED((NPAD, 16), jnp.float32),
        pltpu.VMEM_SHARED((NPAD, DH), jnp.float32),
        pltpu.SemaphoreType.DMA,
        pltpu.SemaphoreType.DMA,
        pltpu.SemaphoreType.DMA,
        pltpu.SemaphoreType.DMA,
    ]

    @functools.partial(pl.kernel, out_type=out_type, mesh=mesh,
                       scratch_types=scratch_types,
                       compiler_params=pltpu.CompilerParams(
                           use_tc_tiling_on_sc=False))
    def edge_kernel(s1t, s2t, htf, srcg, dstg, outw, outh,
                    src_v, dst_v, d2b, s2i, s1b, s2b, hb, wb, whb,
                    accw, acch, g0, g1, t0, t1):
        c = lax.axis_index("c")
        s = lax.axis_index("s")
        gsem = (g0, g1)
        tsem = (t0, t1)

        def zb(i, _):
            wb[i, :] = jnp.zeros((16,), jnp.float32)
            for kk in range(G):
                whb[i, pl.ds(16 * kk, 16)] = jnp.zeros((16,), jnp.float32)
            return 0
        lax.fori_loop(0, KB, zb, 0)
        base = s * RPT
        nfull = RPT // KB
        rem = RPT - nfull * KB
        for i in range(nfull):
            pltpu.sync_copy(wb.at[pl.ds(0, KB)],
                            accw.at[pl.ds(base + KB * i, KB)])
            pltpu.sync_copy(whb.at[pl.ds(0, KB)],
                            acch.at[pl.ds(base + KB * i, KB)])
        if rem:
            pltpu.sync_copy(wb.at[pl.ds(0, rem)],
                            accw.at[pl.ds(base + KB * nfull, rem)])
            pltpu.sync_copy(whb.at[pl.ds(0, rem)],
                            acch.at[pl.ds(base + KB * nfull, rem)])
        plsc.subcore_barrier()

        pltpu.sync_copy(srcg.at[s], src_v)
        pltpu.sync_copy(dstg.at[s], dst_v)
        off = c * NPAD

        def issue_g(b, slot):
            sl = pl.ds(slot * KB, KB)
            for kk in range(KB // 16):
                ds16 = pl.ds(16 * kk, 16)
                d2b[slot, ds16] = dst_v[b, ds16] + off
                s2i[slot, ds16] = src_v[b, ds16] + off
            pltpu.async_copy(s1t.at[s2i.at[slot]], s1b.at[sl], gsem[slot])
            pltpu.async_copy(s2t.at[d2b.at[slot]], s2b.at[sl], gsem[slot])
            pltpu.async_copy(htf.at[d2b.at[slot]], hb.at[sl], gsem[slot])

        def wait_g(slot):
            sl = pl.ds(slot * KB, KB)
            pltpu.make_async_copy(s1t.at[s2i.at[0]], s1b.at[sl],
                                  gsem[slot]).wait()
            pltpu.make_async_copy(s2t.at[d2b.at[0]], s2b.at[sl],
                                  gsem[slot]).wait()
            pltpu.make_async_copy(htf.at[d2b.at[0]], hb.at[sl],
                                  gsem[slot]).wait()

        def issue_s(b, slot):
            sl = pl.ds(slot * KB, KB)
            pltpu.async_copy(wb.at[sl], accw.at[src_v.at[b]], tsem[slot],
                             add=True)
            pltpu.async_copy(whb.at[sl], acch.at[src_v.at[b]], tsem[slot],
                             add=True)

        def wait_s(slot):
            sl = pl.ds(slot * KB, KB)
            pltpu.make_async_copy(wb.at[sl], accw.at[src_v.at[0]],
                                  tsem[slot]).wait()
            pltpu.make_async_copy(whb.at[sl], acch.at[src_v.at[0]],
                                  tsem[slot]).wait()

        def compute(slot):
            @plsc.parallel_loop(0, KB, unroll=16)
            def _(j):
                row = slot * KB + j
                sc_ = s1b[row, :] + s2b[row, :]
                w = jnp.exp(-jnp.where(sc_ >= 0.0, sc_, ALPHA * sc_))
                wb[row, :] = w
                for kk in range(G):
                    whb[row, pl.ds(16 * kk, 16)] = (
                        w[kk] * hb[row, pl.ds(16 * kk, 16)])

        issue_g(0, 0)

        def obody(i, _):
            b0 = 2 * i
            b1 = 2 * i + 1
            issue_g(b1, 1)
            wait_g(0)

            @pl.when(i > 0)
            def _():
                wait_s(0)
            compute(0)
            issue_s(b0, 0)

            @pl.when(i < NB // 2 - 1)
            def _():
                issue_g(b0 + 2, 0)
            wait_g(1)

            @pl.when(i > 0)
            def _():
                wait_s(1)
            compute(1)
            issue_s(b1, 1)
            return 0

        lax.fori_loop(0, NB // 2, obody, 0)
        wait_s(0)
        wait_s(1)
        plsc.subcore_barrier()

        pltpu.sync_copy(accw.at[pl.ds(base, RPT)],
                        outw.at[c, pl.ds(base, RPT)])
        pltpu.sync_copy(acch.at[pl.ds(base, RPT)],
                        outh.at[c, pl.ds(base, RPT)])

    return edge_kernel


def kernel(Corpus_, batch_inputs, entity_embeddings, edge_list, W, a,
           W_out, a_out):
    f32 = jnp.float32
    wc = jnp.transpose(W, (1, 0, 2)).reshape(NF, D)
    p1 = jnp.einsum("hfj,hj->fh", W, a[:, 0, :NHID])
    p2 = jnp.einsum("hfj,hj->fh", W, a[:, 0, NHID:])
    z8 = jnp.zeros((NF, 8), f32)
    z12 = jnp.zeros((NF, 12), f32)
    p1 = jnp.concatenate([p1, z8, p1[:, 4:8], z12], axis=1)
    p2 = jnp.concatenate([p2, z8, p2[:, 4:8], z12], axis=1)
    q1 = W_out @ a_out[0, :D]
    q2 = W_out @ a_out[0, D:]
    zq = jnp.zeros((D, 8), f32)
    zq12 = jnp.zeros((D, 12), f32)
    q1t = jnp.tile(q1[:, None], (1, 8))
    q2t = jnp.tile(q2[:, None], (1, 8))
    q1 = jnp.concatenate([q1t, zq, q1t[:, :4], zq12], axis=1)
    q2 = jnp.concatenate([q2t, zq, q2t[:, :4], zq12], axis=1)
    r = (jnp.arange(D)[None, :] // NHID ==
         jnp.arange(16)[:, None]).astype(f32)

    xp = jnp.pad(entity_embeddings, ((0, NPAD - N), (0, 0)))
    src = edge_list[0].astype(jnp.int32)
    dst = edge_list[1].astype(jnp.int32)
    padv = jnp.full((EPAD - E,), N, jnp.int32)
    srcg = jnp.concatenate([src, padv]).reshape(NS, NB, KB)
    dstg = jnp.concatenate([dst, padv]).reshape(NS, NB, KB)

    ek = _make_edge_kernel()
    h1, s1, s2 = _mm_a(xp, wc, p1, p2)
    aw1, ah1 = ek(s1.reshape(NC * NPAD, 16), s2.reshape(NC * NPAD, 16),
                  h1.reshape(NC * NPAD, DH), srcg, dstg)
    h2, s1b, s2b = _mm_b(aw1, ah1, r, W_out, q1, q2)
    aw2, ah2 = ek(s1b.reshape(NC * NPAD, 16), s2b.reshape(NC * NPAD, 16),
                  h2.reshape(NC * NPAD, DH), srcg, dstg)
    out = _mm_c(aw2, ah2, r)
    return out[:N]


# speedup vs baseline: 19.0882x; 1.0707x over previous
import functools

import jax
import jax.numpy as jnp
from jax import lax
from jax.experimental import pallas as pl
from jax.experimental.pallas import tpu as pltpu
from jax.experimental.pallas import tpu_sc as plsc

N = 10000
E = 320000
NF = 128
NHID = 16
NHEADS = 8
ALPHA = 0.2
D = NHEADS * NHID

NC = 2
NS = 16
DH = D // NC
G = DH // 16
KB = 64
NB = 320
EPT = NB * KB
EPAD = NS * EPT
NPAD = 10112
RPT = NPAD // NS


def _elu(x):
    return jnp.where(x > 0.0, x, jnp.exp(jnp.minimum(x, 0.0)) - 1.0)


def _mm_a_body(x_ref, wc_ref, p1_ref, p2_ref, h_ref, s1_ref, s2_ref):
    x = x_ref[...]
    h = jnp.dot(x, wc_ref[...], preferred_element_type=jnp.float32)
    h_ref[0] = h[:, :DH]
    h_ref[1] = h[:, DH:]
    s1 = jnp.dot(x, p1_ref[...], preferred_element_type=jnp.float32)
    s1_ref[0] = s1[:, :16]
    s1_ref[1] = s1[:, 16:]
    s2 = jnp.dot(x, p2_ref[...], preferred_element_type=jnp.float32)
    s2_ref[0] = s2[:, :16]
    s2_ref[1] = s2[:, 16:]


def _mm_b_body(aw_ref, ah_ref, r_ref, wo_ref, q1_ref, q2_ref,
               h2_ref, s1_ref, s2_ref):
    aw = aw_ref[0]
    ah = jnp.concatenate([ah_ref[0], ah_ref[1]], axis=1)
    den = jnp.dot(aw, r_ref[...], preferred_element_type=jnp.float32) + 1e-16
    x1 = _elu(ah / den)
    h2 = jnp.dot(x1, wo_ref[...], preferred_element_type=jnp.float32)
    h2_ref[0] = h2[:, :DH]
    h2_ref[1] = h2[:, DH:]
    s1 = jnp.dot(x1, q1_ref[...], preferred_element_type=jnp.float32)
    s1_ref[0] = s1[:, :16]
    s1_ref[1] = s1[:, 16:]
    s2 = jnp.dot(x1, q2_ref[...], preferred_element_type=jnp.float32)
    s2_ref[0] = s2[:, :16]
    s2_ref[1] = s2[:, 16:]


def _mm_c_body(aw_ref, ah_ref, r_ref, out_ref):
    aw = aw_ref[0]
    ah = jnp.concatenate([ah_ref[0], ah_ref[1]], axis=1)
    den = jnp.dot(aw, r_ref[...], preferred_element_type=jnp.float32) + 1e-16
    out_ref[...] = _elu(ah / den)


_RB = 632


def _mm_a(xp, wc, p1, p2):
    g = NPAD // _RB
    return pl.pallas_call(
        _mm_a_body,
        grid=(g,),
        in_specs=[
            pl.BlockSpec((_RB, NF), lambda i: (i, 0)),
            pl.BlockSpec((NF, D), lambda i: (0, 0)),
            pl.BlockSpec((NF, 32), lambda i: (0, 0)),
            pl.BlockSpec((NF, 32), lambda i: (0, 0)),
        ],
        out_specs=[
            pl.BlockSpec((NC, _RB, DH), lambda i: (0, i, 0)),
            pl.BlockSpec((NC, _RB, 16), lambda i: (0, i, 0)),
            pl.BlockSpec((NC, _RB, 16), lambda i: (0, i, 0)),
        ],
        out_shape=[
            jax.ShapeDtypeStruct((NC, NPAD, DH), jnp.float32),
            jax.ShapeDtypeStruct((NC, NPAD, 16), jnp.float32),
            jax.ShapeDtypeStruct((NC, NPAD, 16), jnp.float32),
        ],
    )(xp, wc, p1, p2)


def _mm_b(aw, ah, r, wo, q1, q2):
    g = NPAD // _RB
    return pl.pallas_call(
        _mm_b_body,
        grid=(g,),
        in_specs=[
            pl.BlockSpec((NC, _RB, 16), lambda i: (0, i, 0)),
            pl.BlockSpec((NC, _RB, DH), lambda i: (0, i, 0)),
            pl.BlockSpec((16, D), lambda i: (0, 0)),
            pl.BlockSpec((D, D), lambda i: (0, 0)),
            pl.BlockSpec((D, 32), lambda i: (0, 0)),
            pl.BlockSpec((D, 32), lambda i: (0, 0)),
        ],
        out_specs=[
            pl.BlockSpec((NC, _RB, DH), lambda i: (0, i, 0)),
            pl.BlockSpec((NC, _RB, 16), lambda i: (0, i, 0)),
            pl.BlockSpec((NC, _RB, 16), lambda i: (0, i, 0)),
        ],
        out_shape=[
            jax.ShapeDtypeStruct((NC, NPAD, DH), jnp.float32),
            jax.ShapeDtypeStruct((NC, NPAD, 16), jnp.float32),
            jax.ShapeDtypeStruct((NC, NPAD, 16), jnp.float32),
        ],
    )(aw, ah, r, wo, q1, q2)


def _mm_c(aw, ah, r):
    g = NPAD // _RB
    return pl.pallas_call(
        _mm_c_body,
        grid=(g,),
        in_specs=[
            pl.BlockSpec((NC, _RB, 16), lambda i: (0, i, 0)),
            pl.BlockSpec((NC, _RB, DH), lambda i: (0, i, 0)),
            pl.BlockSpec((16, D), lambda i: (0, 0)),
        ],
        out_specs=pl.BlockSpec((_RB, D), lambda i: (i, 0)),
        out_shape=jax.ShapeDtypeStruct((NPAD, D), jnp.float32),
    )(aw, ah, r)


@functools.cache
def _make_edge_kernel():
    mesh = plsc.VectorSubcoreMesh(
        core_axis_name="c", subcore_axis_name="s",
        num_cores=NC, num_subcores=NS)

    out_type = [
        pltpu.HBM((NC, NPAD, 16), jnp.float32),
        pltpu.HBM((NC, NPAD, DH), jnp.float32),
    ]
    scratch_types = [
        pltpu.VMEM((NB, KB), jnp.int32),
        pltpu.VMEM((NB, KB), jnp.int32),
        pltpu.VMEM((4, KB), jnp.int32),
        pltpu.VMEM((4, KB), jnp.int32),
        pltpu.VMEM((4 * KB, 16), jnp.float32),
        pltpu.VMEM((4 * KB, 16), jnp.float32),
        pltpu.VMEM((4 * KB, DH), jnp.float32),
        pltpu.VMEM((2 * KB, 16), jnp.float32),
        pltpu.VMEM((2 * KB, DH), jnp.float32),
        pltpu.VMEM_SHARED((NPAD, 16), jnp.float32),
        pltpu.VMEM_SHARED((NPAD, DH), jnp.float32),
        pltpu.SemaphoreType.DMA,
        pltpu.SemaphoreType.DMA,
        pltpu.SemaphoreType.DMA,
        pltpu.SemaphoreType.DMA,
        pltpu.SemaphoreType.DMA,
        pltpu.SemaphoreType.DMA,
    ]

    @functools.partial(pl.kernel, out_type=out_type, mesh=mesh,
                       scratch_types=scratch_types,
                       compiler_params=pltpu.CompilerParams(
                           use_tc_tiling_on_sc=False))
    def edge_kernel(s1t, s2t, htf, srcg, dstg, outw, outh,
                    src_v, dst_v, d2b, s2i, s1b, s2b, hb, wb, whb,
                    accw, acch, g0, g1, g2, g3, t0, t1):
        c = lax.axis_index("c")
        s = lax.axis_index("s")
        gsem = (g0, g1, g2, g3)
        tsem = (t0, t1)

        def zb(i, _):
            wb[i, :] = jnp.zeros((16,), jnp.float32)
            for kk in range(G):
                whb[i, pl.ds(16 * kk, 16)] = jnp.zeros((16,), jnp.float32)
            return 0
        lax.fori_loop(0, KB, zb, 0)
        base = s * RPT
        nfull = RPT // KB
        rem = RPT - nfull * KB
        for i in range(nfull):
            pltpu.sync_copy(wb.at[pl.ds(0, KB)],
                            accw.at[pl.ds(base + KB * i, KB)])
            pltpu.sync_copy(whb.at[pl.ds(0, KB)],
                            acch.at[pl.ds(base + KB * i, KB)])
        if rem:
            pltpu.sync_copy(wb.at[pl.ds(0, rem)],
                            accw.at[pl.ds(base + KB * nfull, rem)])
            pltpu.sync_copy(whb.at[pl.ds(0, rem)],
                            acch.at[pl.ds(base + KB * nfull, rem)])
        plsc.subcore_barrier()

        pltpu.sync_copy(srcg.at[s], src_v)
        pltpu.sync_copy(dstg.at[s], dst_v)
        off = c * NPAD

        def issue_g(b, slot):
            sl = pl.ds(slot * KB, KB)
            for kk in range(KB // 16):
                ds16 = pl.ds(16 * kk, 16)
                d2b[slot, ds16] = dst_v[b, ds16] + off
                s2i[slot, ds16] = src_v[b, ds16] + off
            pltpu.async_copy(s1t.at[s2i.at[slot]], s1b.at[sl], gsem[slot])
            pltpu.async_copy(s2t.at[d2b.at[slot]], s2b.at[sl], gsem[slot])
            pltpu.async_copy(htf.at[d2b.at[slot]], hb.at[sl], gsem[slot])

        def wait_g(slot):
            sl = pl.ds(slot * KB, KB)
            pltpu.make_async_copy(s1t.at[s2i.at[0]], s1b.at[sl],
                                  gsem[slot]).wait()
            pltpu.make_async_copy(s2t.at[d2b.at[0]], s2b.at[sl],
                                  gsem[slot]).wait()
            pltpu.make_async_copy(htf.at[d2b.at[0]], hb.at[sl],
                                  gsem[slot]).wait()

        def issue_s(b, slot):
            sl = pl.ds(slot * KB, KB)
            pltpu.async_copy(wb.at[sl], accw.at[src_v.at[b]], tsem[slot],
                             add=True)
            pltpu.async_copy(whb.at[sl], acch.at[src_v.at[b]], tsem[slot],
                             add=True)

        def wait_s(slot):
            sl = pl.ds(slot * KB, KB)
            pltpu.make_async_copy(wb.at[sl], accw.at[src_v.at[0]],
                                  tsem[slot]).wait()
            pltpu.make_async_copy(whb.at[sl], acch.at[src_v.at[0]],
                                  tsem[slot]).wait()

        def compute(gslot, cslot):
            @plsc.parallel_loop(0, KB, unroll=8)
            def _(j):
                grow = gslot * KB + j
                crow = cslot * KB + j
                sc_ = s1b[grow, :] + s2b[grow, :]
                w = jnp.exp(-jnp.where(sc_ >= 0.0, sc_, ALPHA * sc_))
                wb[crow, :] = w
                for kk in range(G):
                    whb[crow, pl.ds(16 * kk, 16)] = (
                        w[kk] * hb[crow, pl.ds(16 * kk, 16)])

        issue_g(0, 0)
        issue_g(1, 1)
        issue_g(2, 2)

        def qbody(q, _):
            for u in range(4):
                b = 4 * q + u
                gslot = u
                cslot = u % 2
                if u == 0:
                    issue_g(b + 3, 3)
                else:
                    @pl.when(q < NB // 4 - 1)
                    def _():
                        issue_g(b + 3, (u + 3) % 4)
                wait_g(gslot)
                if u < 2:
                    @pl.when(q > 0)
                    def _():
                        wait_s(cslot)
                else:
                    wait_s(cslot)
                compute(gslot, cslot)
                issue_s(b, cslot)
            return 0

        lax.fori_loop(0, NB // 4, qbody, 0)
        wait_s(0)
        wait_s(1)
        plsc.subcore_barrier()

        pltpu.sync_copy(accw.at[pl.ds(base, RPT)],
                        outw.at[c, pl.ds(base, RPT)])
        pltpu.sync_copy(acch.at[pl.ds(base, RPT)],
                        outh.at[c, pl.ds(base, RPT)])

    return edge_kernel


def kernel(Corpus_, batch_inputs, entity_embeddings, edge_list, W, a,
           W_out, a_out):
    f32 = jnp.float32
    wc = jnp.transpose(W, (1, 0, 2)).reshape(NF, D)
    p1 = jnp.einsum("hfj,hj->fh", W, a[:, 0, :NHID])
    p2 = jnp.einsum("hfj,hj->fh", W, a[:, 0, NHID:])
    z8 = jnp.zeros((NF, 8), f32)
    z12 = jnp.zeros((NF, 12), f32)
    p1 = jnp.concatenate([p1, z8, p1[:, 4:8], z12], axis=1)
    p2 = jnp.concatenate([p2, z8, p2[:, 4:8], z12], axis=1)
    q1 = W_out @ a_out[0, :D]
    q2 = W_out @ a_out[0, D:]
    zq = jnp.zeros((D, 8), f32)
    zq12 = jnp.zeros((D, 12), f32)
    q1t = jnp.tile(q1[:, None], (1, 8))
    q2t = jnp.tile(q2[:, None], (1, 8))
    q1 = jnp.concatenate([q1t, zq, q1t[:, :4], zq12], axis=1)
    q2 = jnp.concatenate([q2t, zq, q2t[:, :4], zq12], axis=1)
    r = (jnp.arange(D)[None, :] // NHID ==
         jnp.arange(16)[:, None]).astype(f32)

    xp = jnp.pad(entity_embeddings, ((0, NPAD - N), (0, 0)))
    src = edge_list[0].astype(jnp.int32)
    dst = edge_list[1].astype(jnp.int32)
    padv = jnp.full((EPAD - E,), N, jnp.int32)
    srcg = jnp.concatenate([src, padv]).reshape(NS, NB, KB)
    dstg = jnp.concatenate([dst, padv]).reshape(NS, NB, KB)

    ek = _make_edge_kernel()
    h1, s1, s2 = _mm_a(xp, wc, p1, p2)
    aw1, ah1 = ek(s1.reshape(NC * NPAD, 16), s2.reshape(NC * NPAD, 16),
                  h1.reshape(NC * NPAD, DH), srcg, dstg)
    h2, s1b, s2b = _mm_b(aw1, ah1, r, W_out, q1, q2)
    aw2, ah2 = ek(s1b.reshape(NC * NPAD, 16), s2b.reshape(NC * NPAD, 16),
                  h2.reshape(NC * NPAD, DH), srcg, dstg)
    out = _mm_c(aw2, ah2, r)
    return out[:N]
